# baseline (speedup 1.0000x reference)
import functools

import jax
import jax.numpy as jnp
from jax.experimental import pallas as pl
from jax.experimental.pallas import tpu as pltpu

_MASK_NEG = -9e15


def _matmul_bias_proj_kernel(m_ref, xw_ref, b_ref, w_ref, o_ref, *, nk, kb):
    k = pl.program_id(1)

    @pl.when(k == 0)
    def _():
        o_ref[...] = jnp.zeros_like(o_ref)

    o_ref[...] += jax.lax.dot_general(
        m_ref[...], xw_ref[pl.ds(k * kb, kb), :],
        (((0,), (0,)), ((), ())), preferred_element_type=jnp.float32)

    @pl.when(k == nk - 1)
    def _():
        o_ref[...] = jnp.dot(o_ref[...] + b_ref[...], w_ref[...],
                             preferred_element_type=jnp.float32)


def _matmul_bias_relu_kernel(a_ref, y_ref, b_ref, o_ref, *, nk, kb):
    k = pl.program_id(1)

    @pl.when(k == 0)
    def _():
        o_ref[...] = jnp.zeros_like(o_ref)

    o_ref[...] += jnp.dot(a_ref[...], y_ref[pl.ds(k * kb, kb), :],
                          preferred_element_type=jnp.float32)

    @pl.when(k == nk - 1)
    def _():
        o_ref[...] = jnp.maximum(o_ref[...] + b_ref[...], 0.0)


def _matmul_kernel(m_ref, z_ref, o_ref, *, kb):
    k = pl.program_id(1)

    @pl.when(k == 0)
    def _():
        o_ref[...] = jnp.zeros_like(o_ref)

    o_ref[...] += jnp.dot(m_ref[...], z_ref[pl.ds(k * kb, kb), :],
                          preferred_element_type=jnp.float32)


def _wh_feats_kernel(x_ref, x1_ref, a_ref, b_ref, src_ref, dst_ref,
                     wh_ref, f1_ref, f2t_ref):
    wh = jnp.dot(x_ref[...], a_ref[...], preferred_element_type=jnp.float32)
    wh = wh + jnp.dot(x1_ref[...], b_ref[...],
                      preferred_element_type=jnp.float32)
    wh_ref[...] = wh
    f1_ref[...] = jnp.dot(wh, src_ref[...], preferred_element_type=jnp.float32)
    f2t_ref[...] = jax.lax.dot_general(
        dst_ref[...], wh, (((0,), (1,)), ((), ())),
        preferred_element_type=jnp.float32)


def _gat_flash_kernel(adj_ref, f1_ref, f2t_ref, wh_ref, o_ref,
                      m_scr, l_scr, acc_scr, *, nj, jb, nheads, hd,
                      final_softmax, nclass):
    j = pl.program_id(1)

    @pl.when(j == 0)
    def _():
        m_scr[...] = jnp.full_like(m_scr, -1e38)
        l_scr[...] = jnp.zeros_like(l_scr)
        acc_scr[...] = jnp.zeros_like(acc_scr)

    adj_pos = adj_ref[...] > 0.0
    for h in range(nheads):
        e = f1_ref[:, h:h + 1] + f2t_ref[h:h + 1, :]
        e = jnp.where(e >= 0.0, e, 0.2 * e)
        e = jnp.where(adj_pos, e, _MASK_NEG)
        m_old = m_scr[:, h:h + 1]
        m_new = jnp.maximum(m_old, jnp.max(e, axis=1, keepdims=True))
        scale = jnp.exp(m_old - m_new)
        p = jnp.exp(e - m_new)
        l_scr[:, h:h + 1] = (l_scr[:, h:h + 1] * scale
                             + jnp.sum(p, axis=1, keepdims=True))
        wh_h = wh_ref[pl.ds(j * jb, jb), h * hd:(h + 1) * hd]
        acc_scr[:, h * hd:(h + 1) * hd] = (
            acc_scr[:, h * hd:(h + 1) * hd] * scale
            + jnp.dot(p, wh_h, preferred_element_type=jnp.float32))
        m_scr[:, h:h + 1] = m_new

    @pl.when(j == nj - 1)
    def _():
        if not final_softmax:
            for h in range(nheads):
                a = acc_scr[:, h * hd:(h + 1) * hd] / l_scr[:, h:h + 1]
                o_ref[:, h * hd:(h + 1) * hd] = jnp.where(
                    a > 0.0, a, jnp.exp(a) - 1.0)
        else:
            a = acc_scr[...] / l_scr[:, 0:1]
            a = jnp.where(a > 0.0, a, jnp.exp(a) - 1.0)
            lane = jax.lax.broadcasted_iota(jnp.int32, a.shape, 1)
            valid = lane < nclass
            am = jnp.where(valid, a, -jnp.inf)
            mx = jnp.max(am, axis=1, keepdims=True)
            s = jnp.sum(jnp.where(valid, jnp.exp(a - mx), 0.0),
                        axis=1, keepdims=True)
            res = a - mx - jnp.log(s)
            o_ref[...] = res[:, :nclass]


def kernel(x, adj, adj_e, M_guanlian, adj_location, W0, b0, W1, b1,
           Wg1, a_src1, a_dst1, Wg2, a_src2, a_dst2):
    del adj_location
    n, nfeat = x.shape
    ne = adj_e.shape[0]
    nedge = W0.shape[1]
    nhid = W1.shape[1]
    nheads = Wg1.shape[0]
    nclass = Wg2.shape[2]
    f32 = jnp.float32

    nfeat_p = (nfeat + 127) // 128 * 128
    xp = jnp.pad(x, ((0, 0), (0, nfeat_p - nfeat)))
    W0p = jnp.pad(W0, ((0, nfeat_p - nfeat), (0, 0)))
    b0r = b0.reshape(1, nedge)
    b1r = b1.reshape(1, nhid)

    hcat = nheads * nhid
    wg1_flat = jnp.transpose(Wg1, (1, 0, 2)).reshape(nfeat + nhid, hcat)
    A1 = jnp.pad(wg1_flat[:nfeat], ((0, nfeat_p - nfeat), (0, 0)))
    B1 = wg1_flat[nfeat:]
    eye = jnp.eye(nheads, dtype=f32)
    src_bd1 = jnp.pad(
        (eye[:, None, :] * a_src1[:, :, None]).reshape(hcat, nheads),
        ((0, 0), (0, 8 - nheads)))
    dst_bd1 = jnp.pad(
        (eye[:, None, :] * a_dst1[:, :, None]).reshape(hcat, nheads),
        ((0, 0), (0, 8 - nheads)))

    nclass_p = 128
    W2p = jnp.pad(Wg2[0], ((0, 0), (0, nclass_p - nclass)))
    src2 = jnp.zeros((nclass_p, 8), f32).at[:nclass, 0].set(a_src2[0])
    dst2 = jnp.zeros((nclass_p, 8), f32).at[:nclass, 0].set(a_dst2[0])

    rb = 512
    xw = pl.pallas_call(
        lambda x_ref, w_ref, o_ref: o_ref.__setitem__(
            ..., jnp.dot(x_ref[...], w_ref[...],
                         preferred_element_type=f32)),
        grid=(n // rb,),
        in_specs=[pl.BlockSpec((rb, nfeat_p), lambda i: (i, 0)),
                  pl.BlockSpec((nfeat_p, nedge), lambda i: (0, 0))],
        out_specs=pl.BlockSpec((rb, nedge), lambda i: (i, 0)),
        out_shape=jax.ShapeDtypeStruct((n, nedge), f32),
    )(xp, W0p)

    eb, kb = 512, 1024
    nk = n // kb
    y1 = pl.pallas_call(
        functools.partial(_matmul_bias_proj_kernel, nk=nk, kb=kb),
        grid=(ne // eb, nk),
        in_specs=[pl.BlockSpec((kb, eb), lambda e, k: (k, e)),
                  pl.BlockSpec((n, nedge), lambda e, k: (0, 0)),
                  pl.BlockSpec((1, nedge), lambda e, k: (0, 0)),
                  pl.BlockSpec((nedge, nhid), lambda e, k: (0, 0))],
        out_specs=pl.BlockSpec((eb, nhid), lambda e, k: (e, 0)),
        out_shape=jax.ShapeDtypeStruct((ne, nhid), f32),
    )(M_guanlian, xw, b0r, W1)

    ib, kb3 = 512, 1024
    nk3 = ne // kb3
    z1 = pl.pallas_call(
        functools.partial(_matmul_bias_relu_kernel, nk=nk3, kb=kb3),
        grid=(ne // ib, nk3),
        in_specs=[pl.BlockSpec((ib, kb3), lambda i, k: (i, k)),
                  pl.BlockSpec((ne, nhid), lambda i, k: (0, 0)),
                  pl.BlockSpec((1, nhid), lambda i, k: (0, 0))],
        out_specs=pl.BlockSpec((ib, nhid), lambda i, k: (i, 0)),
        out_shape=jax.ShapeDtypeStruct((ne, nhid), f32),
    )(adj_e, y1, b1r)

    x1 = pl.pallas_call(
        functools.partial(_matmul_kernel, kb=kb3),
        grid=(n // ib, nk3),
        in_specs=[pl.BlockSpec((ib, kb3), lambda i, k: (i, k)),
                  pl.BlockSpec((ne, nhid), lambda i, k: (0, 0))],
        out_specs=pl.BlockSpec((ib, nhid), lambda i, k: (i, 0)),
        out_shape=jax.ShapeDtypeStruct((n, nhid), f32),
    )(M_guanlian, z1)

    wh1, f1_1, f2t_1 = pl.pallas_call(
        _wh_feats_kernel,
        grid=(n // rb,),
        in_specs=[pl.BlockSpec((rb, nfeat_p), lambda i: (i, 0)),
                  pl.BlockSpec((rb, nhid), lambda i: (i, 0)),
                  pl.BlockSpec((nfeat_p, hcat), lambda i: (0, 0)),
                  pl.BlockSpec((nhid, hcat), lambda i: (0, 0)),
                  pl.BlockSpec((hcat, 8), lambda i: (0, 0)),
                  pl.BlockSpec((hcat, 8), lambda i: (0, 0))],
        out_specs=[pl.BlockSpec((rb, hcat), lambda i: (i, 0)),
                   pl.BlockSpec((rb, 8), lambda i: (i, 0)),
                   pl.BlockSpec((8, rb), lambda i: (0, i))],
        out_shape=[jax.ShapeDtypeStruct((n, hcat), f32),
                   jax.ShapeDtypeStruct((n, 8), f32),
                   jax.ShapeDtypeStruct((8, n), f32)],
    )(xp, x1, A1, B1, src_bd1, dst_bd1)

    aib, ajb = 256, 512
    nj = n // ajb
    h1 = pl.pallas_call(
        functools.partial(_gat_flash_kernel, nj=nj, jb=ajb, nheads=nheads,
                          hd=nhid, final_softmax=False, nclass=nclass),
        grid=(n // aib, nj),
        in_specs=[pl.BlockSpec((aib, ajb), lambda i, j: (i, j)),
                  pl.BlockSpec((aib, 8), lambda i, j: (i, 0)),
                  pl.BlockSpec((8, ajb), lambda i, j: (0, j)),
                  pl.BlockSpec((n, hcat), lambda i, j: (0, 0))],
        out_specs=pl.BlockSpec((aib, hcat), lambda i, j: (i, 0)),
        out_shape=jax.ShapeDtypeStruct((n, hcat), f32),
        scratch_shapes=[pltpu.VMEM((aib, 8), f32),
                        pltpu.VMEM((aib, 8), f32),
                        pltpu.VMEM((aib, hcat), f32)],
    )(adj, f1_1, f2t_1, wh1)

    wh2, f1_2, f2t_2 = pl.pallas_call(
        _wh_feats_kernel,
        grid=(n // rb,),
        in_specs=[pl.BlockSpec((rb, hcat), lambda i: (i, 0)),
                  pl.BlockSpec((rb, nhid), lambda i: (i, 0)),
                  pl.BlockSpec((hcat, nclass_p), lambda i: (0, 0)),
                  pl.BlockSpec((nhid, nclass_p), lambda i: (0, 0)),
                  pl.BlockSpec((nclass_p, 8), lambda i: (0, 0)),
                  pl.BlockSpec((nclass_p, 8), lambda i: (0, 0))],
        out_specs=[pl.BlockSpec((rb, nclass_p), lambda i: (i, 0)),
                   pl.BlockSpec((rb, 8), lambda i: (i, 0)),
                   pl.BlockSpec((8, rb), lambda i: (0, i))],
        out_shape=[jax.ShapeDtypeStruct((n, nclass_p), f32),
                   jax.ShapeDtypeStruct((n, 8), f32),
                   jax.ShapeDtypeStruct((8, n), f32)],
    )(h1, jnp.zeros((n, nhid), f32), W2p, jnp.zeros((nhid, nclass_p), f32),
      src2, dst2)

    out = pl.pallas_call(
        functools.partial(_gat_flash_kernel, nj=nj, jb=ajb, nheads=1,
                          hd=nclass_p, final_softmax=True, nclass=nclass),
        grid=(n // aib, nj),
        in_specs=[pl.BlockSpec((aib, ajb), lambda i, j: (i, j)),
                  pl.BlockSpec((aib, 8), lambda i, j: (i, 0)),
                  pl.BlockSpec((8, ajb), lambda i, j: (0, j)),
                  pl.BlockSpec((n, nclass_p), lambda i, j: (0, 0))],
        out_specs=pl.BlockSpec((aib, nclass), lambda i, j: (i, 0)),
        out_shape=jax.ShapeDtypeStruct((n, nclass), f32),
        scratch_shapes=[pltpu.VMEM((aib, 8), f32),
                        pltpu.VMEM((aib, 8), f32),
                        pltpu.VMEM((aib, nclass_p), f32)],
    )(adj, f1_2, f2t_2, wh2)

    return out


# speedup vs baseline: 2.5206x; 2.5206x over previous
import functools

import jax
import jax.numpy as jnp
from jax.experimental import pallas as pl
from jax.experimental.pallas import tpu as pltpu

_MASK_NEG = -9e15


def _xw_kernel(x_ref, w_ref, o_ref):
    o_ref[...] = jnp.dot(x_ref[...], w_ref[...],
                         preferred_element_type=jnp.float32
                         ).astype(jnp.bfloat16)


def _z0y1_kernel(m_ref, xw_ref, b_ref, w_ref, o_ref, acc_ref, *, nk, kb):
    k = pl.program_id(0)

    @pl.when(k == 0)
    def _():
        acc_ref[...] = jnp.zeros_like(acc_ref)

    acc_ref[...] += jax.lax.dot_general(
        m_ref[...].astype(jnp.bfloat16), xw_ref[pl.ds(k * kb, kb), :],
        (((0,), (0,)), ((), ())), preferred_element_type=jnp.float32)

    @pl.when(k == nk - 1)
    def _():
        o_ref[...] = jnp.dot(acc_ref[...] + b_ref[...], w_ref[...],
                             preferred_element_type=jnp.float32
                             ).astype(jnp.bfloat16)


def _z1_kernel(a_ref, y_ref, b_ref, o_ref):
    o_ref[...] = jnp.maximum(
        jnp.dot(a_ref[...].astype(jnp.bfloat16), y_ref[...],
                preferred_element_type=jnp.float32) + b_ref[...],
        0.0).astype(jnp.bfloat16)


def _x1_kernel(m_ref, z_ref, o_ref):
    o_ref[...] = jnp.dot(m_ref[...].astype(jnp.bfloat16), z_ref[...],
                         preferred_element_type=jnp.float32)


def _wh_feats_kernel(x_ref, x1_ref, a_ref, b_ref, src_ref, dst_ref, one_ref,
                     wh_ref, f1_ref, f2t_ref):
    wh = jnp.dot(x_ref[...], a_ref[...], preferred_element_type=jnp.float32)
    wh = wh + jnp.dot(x1_ref[...], b_ref[...],
                      preferred_element_type=jnp.float32)
    wh_ref[...] = (wh + one_ref[...]).astype(jnp.bfloat16)
    f1_ref[...] = jnp.dot(wh, src_ref[...], preferred_element_type=jnp.float32)
    f2t_ref[...] = jax.lax.dot_general(
        dst_ref[...], wh, (((0,), (1,)), ((), ())),
        preferred_element_type=jnp.float32)


def _gat1_kernel(adj_ref, f1_ref, f2t_ref, wh_ref, o_ref, *, nheads, hd):
    adj_pos = adj_ref[...] > 0.0
    ones_col = jnp.ones((f2t_ref.shape[1], 1), jnp.bfloat16)
    for h in range(nheads):
        e = f1_ref[:, h:h + 1] + f2t_ref[h:h + 1, :]
        e = jnp.maximum(e, 0.2 * e)
        e = jnp.where(adj_pos, e, _MASK_NEG)
        p = jnp.exp(e).astype(jnp.bfloat16)
        acc = jnp.dot(p, wh_ref[:, h * hd:(h + 1) * hd],
                      preferred_element_type=jnp.float32)
        l = jnp.dot(p, ones_col, preferred_element_type=jnp.float32)
        a = acc / l
        o_ref[:, h * hd:(h + 1) * hd] = jnp.where(
            a > 0.0, a, jnp.exp(a) - 1.0)


def _gat2_kernel(adj_ref, f1_ref, f2t_ref, wh_ref, o_ref, *, nclass):
    adj_pos = adj_ref[...] > 0.0
    e = f1_ref[:, 0:1] + f2t_ref[0:1, :]
    e = jnp.maximum(e, 0.2 * e)
    e = jnp.where(adj_pos, e, _MASK_NEG)
    p = jnp.exp(e).astype(jnp.bfloat16)
    acc = jnp.dot(p, wh_ref[...], preferred_element_type=jnp.float32)
    a = acc / acc[:, nclass:nclass + 1]
    a = jnp.where(a > 0.0, a, jnp.exp(a) - 1.0)
    lane = jax.lax.broadcasted_iota(jnp.int32, a.shape, 1)
    valid = lane < nclass
    am = jnp.where(valid, a, -jnp.inf)
    mx = jnp.max(am, axis=1, keepdims=True)
    s = jnp.sum(jnp.where(valid, jnp.exp(a - mx), 0.0),
                axis=1, keepdims=True)
    res = a - mx - jnp.log(s)
    o_ref[...] = res[:, :nclass]


def kernel(x, adj, adj_e, M_guanlian, adj_location, W0, b0, W1, b1,
           Wg1, a_src1, a_dst1, Wg2, a_src2, a_dst2):
    del adj_location
    n, nfeat = x.shape
    ne = adj_e.shape[0]
    nedge = W0.shape[1]
    nhid = W1.shape[1]
    nheads = Wg1.shape[0]
    nclass = Wg2.shape[2]
    f32 = jnp.float32
    bf16 = jnp.bfloat16

    nfeat_p = (nfeat + 127) // 128 * 128
    xp = jnp.pad(x, ((0, 0), (0, nfeat_p - nfeat)))
    W0p = jnp.pad(W0, ((0, nfeat_p - nfeat), (0, 0)))
    b0r = b0.reshape(1, nedge)
    b1r = b1.reshape(1, nhid)

    hcat = nheads * nhid
    wg1_flat = jnp.transpose(Wg1, (1, 0, 2)).reshape(nfeat + nhid, hcat)
    A1 = jnp.pad(wg1_flat[:nfeat], ((0, nfeat_p - nfeat), (0, 0)))
    B1 = wg1_flat[nfeat:]
    eye = jnp.eye(nheads, dtype=f32)
    src_bd1 = jnp.pad(
        (eye[:, None, :] * a_src1[:, :, None]).reshape(hcat, nheads),
        ((0, 0), (0, 8 - nheads)))
    dst_bd1 = jnp.pad(
        (eye[:, None, :] * a_dst1[:, :, None]).reshape(hcat, nheads),
        ((0, 0), (0, 8 - nheads)))
    zeros_row1 = jnp.zeros((1, hcat), f32)

    W2a = jnp.zeros((hcat, 8), f32).at[:, :nclass].set(Wg2[0])
    one_row2 = jnp.zeros((1, 8), f32).at[0, nclass].set(1.0)
    src2 = jnp.zeros((8, 8), f32).at[:nclass, 0].set(a_src2[0])
    dst2 = jnp.zeros((8, 8), f32).at[:nclass, 0].set(a_dst2[0])

    rb = 512
    xw = pl.pallas_call(
        _xw_kernel,
        grid=(n // rb,),
        in_specs=[pl.BlockSpec((rb, nfeat_p), lambda i: (i, 0)),
                  pl.BlockSpec((nfeat_p, nedge), lambda i: (0, 0))],
        out_specs=pl.BlockSpec((rb, nedge), lambda i: (i, 0)),
        out_shape=jax.ShapeDtypeStruct((n, nedge), bf16),
    )(xp, W0p)

    kb = 512
    nk = n // kb
    y1 = pl.pallas_call(
        functools.partial(_z0y1_kernel, nk=nk, kb=kb),
        grid=(nk,),
        in_specs=[pl.BlockSpec((kb, ne), lambda k: (k, 0)),
                  pl.BlockSpec((n, nedge), lambda k: (0, 0)),
                  pl.BlockSpec((1, nedge), lambda k: (0, 0)),
                  pl.BlockSpec((nedge, nhid), lambda k: (0, 0))],
        out_specs=pl.BlockSpec((ne, nhid), lambda k: (0, 0)),
        out_shape=jax.ShapeDtypeStruct((ne, nhid), bf16),
        scratch_shapes=[pltpu.VMEM((ne, nedge), f32)],
    )(M_guanlian, xw, b0r, W1)

    ib = 512
    z1 = pl.pallas_call(
        _z1_kernel,
        grid=(ne // ib,),
        in_specs=[pl.BlockSpec((ib, ne), lambda i: (i, 0)),
                  pl.BlockSpec((ne, nhid), lambda i: (0, 0)),
                  pl.BlockSpec((1, nhid), lambda i: (0, 0))],
        out_specs=pl.BlockSpec((ib, nhid), lambda i: (i, 0)),
        out_shape=jax.ShapeDtypeStruct((ne, nhid), bf16),
    )(adj_e, y1, b1r)

    x1 = pl.pallas_call(
        _x1_kernel,
        grid=(n // ib,),
        in_specs=[pl.BlockSpec((ib, ne), lambda i: (i, 0)),
                  pl.BlockSpec((ne, nhid), lambda i: (0, 0))],
        out_specs=pl.BlockSpec((ib, nhid), lambda i: (i, 0)),
        out_shape=jax.ShapeDtypeStruct((n, nhid), f32),
    )(M_guanlian, z1)

    wh1, f1_1, f2t_1 = pl.pallas_call(
        _wh_feats_kernel,
        grid=(n // rb,),
        in_specs=[pl.BlockSpec((rb, nfeat_p), lambda i: (i, 0)),
                  pl.BlockSpec((rb, nhid), lambda i: (i, 0)),
                  pl.BlockSpec((nfeat_p, hcat), lambda i: (0, 0)),
                  pl.BlockSpec((nhid, hcat), lambda i: (0, 0)),
                  pl.BlockSpec((hcat, 8), lambda i: (0, 0)),
                  pl.BlockSpec((hcat, 8), lambda i: (0, 0)),
                  pl.BlockSpec((1, hcat), lambda i: (0, 0))],
        out_specs=[pl.BlockSpec((rb, hcat), lambda i: (i, 0)),
                   pl.BlockSpec((rb, 8), lambda i: (i, 0)),
                   pl.BlockSpec((8, rb), lambda i: (0, i))],
        out_shape=[jax.ShapeDtypeStruct((n, hcat), bf16),
                   jax.ShapeDtypeStruct((n, 8), f32),
                   jax.ShapeDtypeStruct((8, n), f32)],
    )(xp, x1, A1, B1, src_bd1, dst_bd1, zeros_row1)

    gb = 512
    h1 = pl.pallas_call(
        functools.partial(_gat1_kernel, nheads=nheads, hd=nhid),
        grid=(n // gb,),
        in_specs=[pl.BlockSpec((gb, n), lambda i: (i, 0)),
                  pl.BlockSpec((gb, 8), lambda i: (i, 0)),
                  pl.BlockSpec((8, n), lambda i: (0, 0)),
                  pl.BlockSpec((n, hcat), lambda i: (0, 0))],
        out_specs=pl.BlockSpec((gb, hcat), lambda i: (i, 0)),
        out_shape=jax.ShapeDtypeStruct((n, hcat), f32),
    )(adj, f1_1, f2t_1, wh1)

    wh2, f1_2, f2t_2 = pl.pallas_call(
        _wh_feats_kernel,
        grid=(n // rb,),
        in_specs=[pl.BlockSpec((rb, hcat), lambda i: (i, 0)),
                  pl.BlockSpec((rb, nhid), lambda i: (i, 0)),
                  pl.BlockSpec((hcat, 8), lambda i: (0, 0)),
                  pl.BlockSpec((nhid, 8), lambda i: (0, 0)),
                  pl.BlockSpec((8, 8), lambda i: (0, 0)),
                  pl.BlockSpec((8, 8), lambda i: (0, 0)),
                  pl.BlockSpec((1, 8), lambda i: (0, 0))],
        out_specs=[pl.BlockSpec((rb, 8), lambda i: (i, 0)),
                   pl.BlockSpec((rb, 8), lambda i: (i, 0)),
                   pl.BlockSpec((8, rb), lambda i: (0, i))],
        out_shape=[jax.ShapeDtypeStruct((n, 8), bf16),
                   jax.ShapeDtypeStruct((n, 8), f32),
                   jax.ShapeDtypeStruct((8, n), f32)],
    )(h1, jnp.zeros((n, nhid), f32), W2a, jnp.zeros((nhid, 8), f32),
      src2, dst2, one_row2)

    out = pl.pallas_call(
        functools.partial(_gat2_kernel, nclass=nclass),
        grid=(n // gb,),
        in_specs=[pl.BlockSpec((gb, n), lambda i: (i, 0)),
                  pl.BlockSpec((gb, 8), lambda i: (i, 0)),
                  pl.BlockSpec((8, n), lambda i: (0, 0)),
                  pl.BlockSpec((n, 8), lambda i: (0, 0))],
        out_specs=pl.BlockSpec((gb, nclass), lambda i: (i, 0)),
        out_shape=jax.ShapeDtypeStruct((n, nclass), f32),
    )(adj, f1_2, f2t_2, wh2)

    return out


# speedup vs baseline: 3.1649x; 1.2556x over previous
import functools

import jax
import jax.numpy as jnp
from jax.experimental import pallas as pl
from jax.experimental.pallas import tpu as pltpu


def _y1_kernel(m_ref, x_ref, w0_ref, b_ref, w1_ref, o_ref, acc_ref, *, nk):
    k = pl.program_id(0)

    @pl.when(k == 0)
    def _():
        acc_ref[...] = jnp.zeros_like(acc_ref)

    xw = jax.lax.dot_general(
        x_ref[...].astype(jnp.bfloat16), w0_ref[...],
        (((0,), (0,)), ((), ())),
        preferred_element_type=jnp.float32).astype(jnp.bfloat16)
    acc_ref[...] += jax.lax.dot_general(
        m_ref[...].astype(jnp.bfloat16), xw,
        (((0,), (0,)), ((), ())), preferred_element_type=jnp.float32)

    @pl.when(k == nk - 1)
    def _():
        o_ref[...] = jnp.dot(acc_ref[...] + b_ref[...], w1_ref[...],
                             preferred_element_type=jnp.float32
                             ).astype(jnp.bfloat16)


def _z1_kernel(a_ref, y_ref, b_ref, o_ref):
    o_ref[...] = jnp.maximum(
        jnp.dot(a_ref[...].astype(jnp.bfloat16), y_ref[...],
                preferred_element_type=jnp.float32) + b_ref[...],
        0.0).astype(jnp.bfloat16)


def _x1_wh_kernel(m_ref, z_ref, x_ref, a_ref, b_ref, src_ref, dst_ref,
                  one_ref, wh_ref, f1_ref, f2t_ref):
    x1 = jnp.dot(m_ref[...].astype(jnp.bfloat16), z_ref[...],
                 preferred_element_type=jnp.float32)
    wh = jax.lax.dot_general(
        x_ref[...].astype(jnp.bfloat16), a_ref[...],
        (((0,), (0,)), ((), ())), preferred_element_type=jnp.float32)
    wh = wh + jnp.dot(x1.astype(jnp.bfloat16), b_ref[...],
                      preferred_element_type=jnp.float32)
    wh_ref[...] = (wh + one_ref[...]).astype(jnp.bfloat16)
    f1_ref[...] = jnp.dot(wh, src_ref[...], preferred_element_type=jnp.float32)
    f2t_ref[...] = jax.lax.dot_general(
        dst_ref[...], wh, (((0,), (1,)), ((), ())),
        preferred_element_type=jnp.float32)


def _gat1_kernel(adj_ref, f1_ref, f2t_ref, wh_ref, w2_ref, src2_ref,
                 dst2_ref, one2_ref, wh2_ref, f12_ref, f2t2_ref,
                 *, nheads, hd):
    mask01 = (adj_ref[...] > 0.0).astype(jnp.bfloat16)
    f1b = f1_ref[...].astype(jnp.bfloat16)
    f2tb = f2t_ref[...].astype(jnp.bfloat16)
    h1_parts = []
    for h in range(nheads):
        v = f1b[:, h:h + 1] + f2tb[h:h + 1, :]
        v = jnp.maximum(v, jnp.bfloat16(0.2) * v)
        p = jnp.exp2(v) * mask01
        acc = jnp.dot(p, wh_ref[:, h * 128:(h + 1) * 128],
                      preferred_element_type=jnp.float32)
        a = acc[:, :hd] / acc[:, hd:hd + 1]
        h1_parts.append(jnp.where(a > 0.0, a, jnp.exp(a) - 1.0))
    h1 = jnp.concatenate(h1_parts, axis=1)
    wh2 = jnp.dot(h1.astype(jnp.bfloat16), w2_ref[...],
                  preferred_element_type=jnp.float32)
    wh2_ref[...] = (wh2 + one2_ref[...]).astype(jnp.bfloat16)
    f12_ref[...] = jnp.dot(wh2, src2_ref[...],
                           preferred_element_type=jnp.float32)
    f2t2_ref[...] = jax.lax.dot_general(
        dst2_ref[...], wh2, (((0,), (1,)), ((), ())),
        preferred_element_type=jnp.float32)


def _gat2_kernel(adj_ref, f1_ref, f2t_ref, wh_ref, o_ref, *, nclass):
    mask01 = (adj_ref[...] > 0.0).astype(jnp.bfloat16)
    v = f1_ref[...].astype(jnp.bfloat16)[:, 0:1] \
        + f2t_ref[...].astype(jnp.bfloat16)[0:1, :]
    v = jnp.maximum(v, jnp.bfloat16(0.2) * v)
    p = jnp.exp2(v) * mask01
    acc = jnp.dot(p, wh_ref[...], preferred_element_type=jnp.float32)
    a = acc / acc[:, nclass:nclass + 1]
    a = jnp.where(a > 0.0, a, jnp.exp(a) - 1.0)
    lane = jax.lax.broadcasted_iota(jnp.int32, a.shape, 1)
    valid = lane < nclass
    am = jnp.where(valid, a, -jnp.inf)
    mx = jnp.max(am, axis=1, keepdims=True)
    s = jnp.sum(jnp.where(valid, jnp.exp(a - mx), 0.0),
                axis=1, keepdims=True)
    res = a - mx - jnp.log(s)
    o_ref[...] = res[:, :nclass]


def kernel(x, adj, adj_e, M_guanlian, adj_location, W0, b0, W1, b1,
           Wg1, a_src1, a_dst1, Wg2, a_src2, a_dst2):
    del adj_location
    n, nfeat = x.shape
    ne = adj_e.shape[0]
    nedge = W0.shape[1]
    nhid = W1.shape[1]
    nheads = Wg1.shape[0]
    nclass = Wg2.shape[2]
    f32 = jnp.float32
    bf16 = jnp.bfloat16

    xt = x.T
    b0r = b0.reshape(1, nedge)
    b1r = b1.reshape(1, nhid)
    w0b = W0.astype(bf16)

    hcat = nheads * nhid
    hp = nheads * 128
    wg1_pad = jnp.pad(
        jnp.transpose(Wg1, (1, 0, 2)),
        ((0, 0), (0, 0), (0, 128 - nhid))).reshape(nfeat + nhid, hp)
    A1 = wg1_pad[:nfeat].astype(bf16)
    B1 = wg1_pad[nfeat:].astype(bf16)
    one_row1 = jnp.zeros((1, hp), f32).at[0, nhid::128].set(1.0)
    eye = jnp.eye(nheads, dtype=f32)
    log2e = 1.4426950408889634
    a_src1p = jnp.pad(a_src1, ((0, 0), (0, 128 - nhid)))
    a_dst1p = jnp.pad(a_dst1, ((0, 0), (0, 128 - nhid)))
    src_bd1 = jnp.pad(
        (eye[:, None, :] * a_src1p[:, :, None]).reshape(hp, nheads),
        ((0, 0), (0, 8 - nheads))) * log2e
    dst_bd1 = jnp.pad(
        (eye[:, None, :] * a_dst1p[:, :, None]).reshape(hp, nheads),
        ((0, 0), (0, 8 - nheads))) * log2e

    W2a = jnp.zeros((hcat, 8), f32).at[:, :nclass].set(Wg2[0]).astype(bf16)
    one_row2 = jnp.zeros((1, 8), f32).at[0, nclass].set(1.0)
    src2 = jnp.zeros((8, 8), f32).at[:nclass, 0].set(a_src2[0] * log2e)
    dst2 = jnp.zeros((8, 8), f32).at[:nclass, 0].set(a_dst2[0] * log2e)

    kb = 512
    nk = n // kb
    y1 = pl.pallas_call(
        functools.partial(_y1_kernel, nk=nk),
        grid=(nk,),
        in_specs=[pl.BlockSpec((kb, ne), lambda k: (k, 0)),
                  pl.BlockSpec((nfeat, kb), lambda k: (0, k)),
                  pl.BlockSpec((nfeat, nedge), lambda k: (0, 0)),
                  pl.BlockSpec((1, nedge), lambda k: (0, 0)),
                  pl.BlockSpec((nedge, nhid), lambda k: (0, 0))],
        out_specs=pl.BlockSpec((ne, nhid), lambda k: (0, 0)),
        out_shape=jax.ShapeDtypeStruct((ne, nhid), bf16),
        scratch_shapes=[pltpu.VMEM((ne, nedge), f32)],
    )(M_guanlian, xt, w0b, b0r, W1)

    ib = 512
    z1 = pl.pallas_call(
        _z1_kernel,
        grid=(ne // ib,),
        in_specs=[pl.BlockSpec((ib, ne), lambda i: (i, 0)),
                  pl.BlockSpec((ne, nhid), lambda i: (0, 0)),
                  pl.BlockSpec((1, nhid), lambda i: (0, 0))],
        out_specs=pl.BlockSpec((ib, nhid), lambda i: (i, 0)),
        out_shape=jax.ShapeDtypeStruct((ne, nhid), bf16),
    )(adj_e, y1, b1r)

    wh1, f1_1, f2t_1 = pl.pallas_call(
        _x1_wh_kernel,
        grid=(n // ib,),
        in_specs=[pl.BlockSpec((ib, ne), lambda i: (i, 0)),
                  pl.BlockSpec((ne, nhid), lambda i: (0, 0)),
                  pl.BlockSpec((nfeat, ib), lambda i: (0, i)),
                  pl.BlockSpec((nfeat, hp), lambda i: (0, 0)),
                  pl.BlockSpec((nhid, hp), lambda i: (0, 0)),
                  pl.BlockSpec((hp, 8), lambda i: (0, 0)),
                  pl.BlockSpec((hp, 8), lambda i: (0, 0)),
                  pl.BlockSpec((1, hp), lambda i: (0, 0))],
        out_specs=[pl.BlockSpec((ib, hp), lambda i: (i, 0)),
                   pl.BlockSpec((ib, 8), lambda i: (i, 0)),
                   pl.BlockSpec((8, ib), lambda i: (0, i))],
        out_shape=[jax.ShapeDtypeStruct((n, hp), bf16),
                   jax.ShapeDtypeStruct((n, 8), f32),
                   jax.ShapeDtypeStruct((8, n), f32)],
    )(M_guanlian, z1, xt, A1, B1, src_bd1, dst_bd1, one_row1)

    gb = 512
    wh2, f1_2, f2t_2 = pl.pallas_call(
        functools.partial(_gat1_kernel, nheads=nheads, hd=nhid),
        grid=(n // gb,),
        in_specs=[pl.BlockSpec((gb, n), lambda i: (i, 0)),
                  pl.BlockSpec((gb, 8), lambda i: (i, 0)),
                  pl.BlockSpec((8, n), lambda i: (0, 0)),
                  pl.BlockSpec((n, hp), lambda i: (0, 0)),
                  pl.BlockSpec((hcat, 8), lambda i: (0, 0)),
                  pl.BlockSpec((8, 8), lambda i: (0, 0)),
                  pl.BlockSpec((8, 8), lambda i: (0, 0)),
                  pl.BlockSpec((1, 8), lambda i: (0, 0))],
        out_specs=[pl.BlockSpec((gb, 8), lambda i: (i, 0)),
                   pl.BlockSpec((gb, 8), lambda i: (i, 0)),
                   pl.BlockSpec((8, gb), lambda i: (0, i))],
        out_shape=[jax.ShapeDtypeStruct((n, 8), bf16),
                   jax.ShapeDtypeStruct((n, 8), f32),
                   jax.ShapeDtypeStruct((8, n), f32)],
    )(adj, f1_1, f2t_1, wh1, W2a, src2, dst2, one_row2)

    out = pl.pallas_call(
        functools.partial(_gat2_kernel, nclass=nclass),
        grid=(n // gb,),
        in_specs=[pl.BlockSpec((gb, n), lambda i: (i, 0)),
                  pl.BlockSpec((gb, 8), lambda i: (i, 0)),
                  pl.BlockSpec((8, n), lambda i: (0, 0)),
                  pl.BlockSpec((n, 8), lambda i: (0, 0))],
        out_specs=pl.BlockSpec((gb, nclass), lambda i: (i, 0)),
        out_shape=jax.ShapeDtypeStruct((n, nclass), f32),
    )(adj, f1_2, f2t_2, wh2)

    return out
